# baseline (speedup 1.0000x reference)
import functools

import jax
import jax.numpy as jnp
from jax import lax
from jax.experimental import pallas as pl
from jax.experimental.pallas import tpu as pltpu
from jax.experimental.pallas import tpu_sc as plsc

NC = 2
NS = 16
NW = NC * NS


def _round_up(a, m):
    return -(-a // m) * m


def _plan_edges(e, max_chunk=128):
    for chunk in range(max_chunk, 0, -8):
        if e % (NW * chunk) == 0:
            return chunk, e // (NW * chunk), 0
    chunk = max_chunk
    nch = -(-e // (NW * chunk))
    return chunk, nch, NW * chunk * nch - e


def _sc_degree(dst3, nrows):
    nw, nch, chunk = dst3.shape
    rpt = nrows // NS

    @functools.partial(
        pl.kernel,
        out_type=jax.ShapeDtypeStruct((NC, nrows), jnp.float32),
        mesh=plsc.VectorSubcoreMesh(core_axis_name="c", subcore_axis_name="s"),
        scratch_types=[
            pltpu.VMEM((nch, chunk), jnp.int32),
            pltpu.VMEM((chunk,), jnp.float32),
            pltpu.VMEM((rpt,), jnp.float32),
            pltpu.VMEM_SHARED((nrows,), jnp.float32),
        ],
    )
    def deg_kernel(dst_hbm, degp_hbm, idx_v, ones_v, zb_v, deg_sh):
        c = lax.axis_index("c")
        s = lax.axis_index("s")
        w = c * NS + s
        pltpu.sync_copy(dst_hbm.at[w], idx_v)

        @pl.loop(0, chunk // 16)
        def _ones(i):
            ones_v[pl.ds(i * 16, 16)] = jnp.ones((16,), jnp.float32)

        @pl.loop(0, rpt // 16)
        def _zb(i):
            zb_v[pl.ds(i * 16, 16)] = jnp.zeros((16,), jnp.float32)

        pltpu.sync_copy(zb_v, deg_sh.at[pl.ds(s * rpt, rpt)])
        plsc.subcore_barrier()

        @pl.loop(0, nch)
        def _scat(j):
            pltpu.sync_copy(ones_v, deg_sh.at[idx_v.at[j]], add=True)

        plsc.subcore_barrier()
        pltpu.sync_copy(deg_sh.at[pl.ds(s * rpt, rpt)], zb_v)
        pltpu.sync_copy(zb_v, degp_hbm.at[c, pl.ds(s * rpt, rpt)])

    return deg_kernel(dst3)


def _sc_scatter(g, src3, dst3, nrows):
    nw, nch, chunk = src3.shape
    d = g.shape[1]
    rpt = nrows // NS
    nfull = rpt // chunk
    rem = rpt % chunk

    @functools.partial(
        pl.kernel,
        out_type=jax.ShapeDtypeStruct((NC, nrows, d), jnp.float32),
        mesh=plsc.VectorSubcoreMesh(core_axis_name="c", subcore_axis_name="s"),
        scratch_types=[
            pltpu.VMEM((nch, chunk), jnp.int32),
            pltpu.VMEM((nch, chunk), jnp.int32),
            pltpu.VMEM((chunk, d), jnp.float32),
            pltpu.VMEM_SHARED((nrows, d), jnp.float32),
            pltpu.SemaphoreType.DMA,
        ],
    )
    def scat_kernel(g_hbm, src_hbm, dst_hbm, acc_hbm,
                    src_v, dst_v, buf_a, acc_sh, sem_a):
        c = lax.axis_index("c")
        s = lax.axis_index("s")
        w = c * NS + s
        pltpu.sync_copy(src_hbm.at[w], src_v)
        pltpu.sync_copy(dst_hbm.at[w], dst_v)

        @pl.loop(0, chunk)
        def _zr(r):
            @pl.loop(0, d // 16)
            def _zc(i):
                buf_a[r, pl.ds(i * 16, 16)] = jnp.zeros((16,), jnp.float32)

        base = s * rpt

        @pl.loop(0, nfull)
        def _za(k):
            pltpu.sync_copy(buf_a, acc_sh.at[pl.ds(base + k * chunk, chunk)])

        if rem:
            pltpu.sync_copy(buf_a.at[pl.ds(0, rem)],
                            acc_sh.at[pl.ds(base + nfull * chunk, rem)])

        plsc.subcore_barrier()

        def _gather(j, buf, sem):
            return pltpu.make_async_copy(g_hbm.at[src_v.at[j]], buf, sem)

        def _scatter(j, buf, sem):
            del sem
            pltpu.sync_copy(buf, acc_sh.at[dst_v.at[j]], add=True)

        @pl.loop(0, nch)
        def _edge(j):
            _gather(j, buf_a, sem_a).start()
            _gather(j, buf_a, sem_a).wait()
            _scatter(j, buf_a, sem_a)

        plsc.subcore_barrier()

        @pl.loop(0, nfull)
        def _out(k):
            pltpu.sync_copy(acc_sh.at[pl.ds(base + k * chunk, chunk)], buf_a)
            pltpu.sync_copy(buf_a, acc_hbm.at[c, pl.ds(base + k * chunk, chunk)])

        if rem:
            pltpu.sync_copy(acc_sh.at[pl.ds(base + nfull * chunk, rem)],
                            buf_a.at[pl.ds(0, rem)])
            pltpu.sync_copy(buf_a.at[pl.ds(0, rem)],
                            acc_hbm.at[c, pl.ds(base + nfull * chunk, rem)])

    return scat_kernel(g, src3, dst3)


def _tca_body(x_ref, w_ref, degp_ref, g_ref):
    deg = jnp.sum(degp_ref[...], axis=0) + 1.0
    dinv = lax.rsqrt(deg)
    h = jnp.dot(x_ref[...], w_ref[...], preferred_element_type=jnp.float32)
    g_ref[...] = h * dinv[:, None]


def _tc_transform(x, W, degp):
    n, d_in = x.shape
    d = W.shape[1]
    br = 512
    return pl.pallas_call(
        _tca_body,
        grid=(pl.cdiv(n, br),),
        in_specs=[
            pl.BlockSpec((br, d_in), lambda i: (i, 0)),
            pl.BlockSpec((d_in, d), lambda i: (0, 0)),
            pl.BlockSpec((NC, br), lambda i: (0, i)),
        ],
        out_specs=pl.BlockSpec((br, d), lambda i: (i, 0)),
        out_shape=jax.ShapeDtypeStruct((n, d), jnp.float32),
    )(x, W, degp)


def _tcb_body(accp_ref, g_ref, degp_ref, b_ref, o_ref):
    deg = jnp.sum(degp_ref[...], axis=0) + 1.0
    dinv = lax.rsqrt(deg)
    z = (accp_ref[0] + accp_ref[1] + g_ref[...]) * dinv[:, None] + b_ref[...]
    m = jnp.max(z, axis=1, keepdims=True)
    ez = jnp.exp(z - m)
    o_ref[...] = z - m - jnp.log(jnp.sum(ez, axis=1, keepdims=True))


def _tc_finalize(accp, g, degp, b):
    n, d = g.shape
    br = 512
    return pl.pallas_call(
        _tcb_body,
        grid=(pl.cdiv(n, br),),
        in_specs=[
            pl.BlockSpec((NC, br, d), lambda i: (0, i, 0)),
            pl.BlockSpec((br, d), lambda i: (i, 0)),
            pl.BlockSpec((NC, br), lambda i: (0, i)),
            pl.BlockSpec((1, d), lambda i: (0, 0)),
        ],
        out_specs=pl.BlockSpec((br, d), lambda i: (i, 0)),
        out_shape=jax.ShapeDtypeStruct((n, d), jnp.float32),
    )(accp, g, degp, b.reshape(1, d))


def kernel(x, edge_index, W, b):
    n, d_in = x.shape
    d = W.shape[1]
    e = edge_index.shape[1]
    src = edge_index[0]
    dst = edge_index[1]

    chunk_d, nch_d, pad = _plan_edges(e)
    chunk_s, nch_s, pad_s = _plan_edges(e)
    assert pad == pad_s
    if pad:
        ar = jnp.arange(pad, dtype=jnp.int32)
        src = jnp.concatenate([src, ar % n])
        dst = jnp.concatenate([dst, n + (ar % 64)])
        nrows = _round_up(n + 64, 256)
    else:
        nrows = _round_up(n, 256)

    degp = _sc_degree(dst.reshape(NW, nch_d, chunk_d), nrows)
    g = _tc_transform(x, W, degp)
    accp = _sc_scatter(g, src.reshape(NW, nch_s, chunk_s),
                       dst.reshape(NW, nch_s, chunk_s), nrows)
    out = _tc_finalize(accp, g, degp, b)
    return out


# speedup vs baseline: 1.3800x; 1.3800x over previous
import functools

import jax
import jax.numpy as jnp
from jax import lax
from jax.experimental import pallas as pl
from jax.experimental.pallas import tpu as pltpu
from jax.experimental.pallas import tpu_sc as plsc

NC = 2
NS = 16
NW = NC * NS


def _round_up(a, m):
    return -(-a // m) * m


def _plan_edges(e, max_chunk=128):
    for chunk in range(max_chunk, 0, -8):
        if e % (NW * chunk) == 0:
            return chunk, e // (NW * chunk), 0
    chunk = max_chunk
    nch = -(-e // (NW * chunk))
    return chunk, nch, NW * chunk * nch - e


def _sc_degree(dst3, nrows):
    nw, nch, chunk = dst3.shape
    rpt = nrows // NS

    @functools.partial(
        pl.kernel,
        out_type=jax.ShapeDtypeStruct((NC, nrows), jnp.float32),
        mesh=plsc.VectorSubcoreMesh(core_axis_name="c", subcore_axis_name="s"),
        scratch_types=[
            pltpu.VMEM((nch, chunk), jnp.int32),
            pltpu.VMEM((chunk,), jnp.float32),
            pltpu.VMEM((rpt,), jnp.float32),
            pltpu.VMEM_SHARED((nrows,), jnp.float32),
        ],
    )
    def deg_kernel(dst_hbm, degp_hbm, idx_v, ones_v, zb_v, deg_sh):
        c = lax.axis_index("c")
        s = lax.axis_index("s")
        w = c * NS + s
        pltpu.sync_copy(dst_hbm.at[w], idx_v)

        @pl.loop(0, chunk // 16)
        def _ones(i):
            ones_v[pl.ds(i * 16, 16)] = jnp.ones((16,), jnp.float32)

        @pl.loop(0, rpt // 16)
        def _zb(i):
            zb_v[pl.ds(i * 16, 16)] = jnp.zeros((16,), jnp.float32)

        pltpu.sync_copy(zb_v, deg_sh.at[pl.ds(s * rpt, rpt)])
        plsc.subcore_barrier()

        @pl.loop(0, nch)
        def _scat(j):
            pltpu.sync_copy(ones_v, deg_sh.at[idx_v.at[j]], add=True)

        plsc.subcore_barrier()
        pltpu.sync_copy(deg_sh.at[pl.ds(s * rpt, rpt)], zb_v)
        pltpu.sync_copy(zb_v, degp_hbm.at[c, pl.ds(s * rpt, rpt)])

    return deg_kernel(dst3)


def _sc_scatter(g, src2, dst3, nrows):
    nw, nch, chunk = dst3.shape
    d = g.shape[1]
    rpt = nrows // NS
    nfull = rpt // chunk
    rem = rpt % chunk

    @functools.partial(
        pl.kernel,
        out_type=jax.ShapeDtypeStruct((NC, nrows, d), jnp.float32),
        mesh=plsc.VectorSubcoreMesh(core_axis_name="c", subcore_axis_name="s"),
        scratch_types=[
            pltpu.VMEM((nch * chunk,), jnp.int32),
            pltpu.VMEM((nch, chunk), jnp.int32),
            pltpu.VMEM((chunk, d), jnp.float32),
            pltpu.VMEM((chunk, d), jnp.float32),
            pltpu.VMEM_SHARED((nrows, d), jnp.float32),
            pltpu.SemaphoreType.DMA,
            pltpu.SemaphoreType.DMA,
        ],
    )
    def scat_kernel(g_hbm, src_hbm, dst_hbm, acc_hbm,
                    src_v, dst_v, buf_a, buf_b, acc_sh, sem_a, sem_b):
        c = lax.axis_index("c")
        s = lax.axis_index("s")
        w = c * NS + s
        pltpu.sync_copy(src_hbm.at[w], src_v)
        pltpu.sync_copy(dst_hbm.at[w], dst_v)

        @pl.loop(0, chunk)
        def _zr(r):
            @pl.loop(0, d // 16)
            def _zc(i):
                buf_a[r, pl.ds(i * 16, 16)] = jnp.zeros((16,), jnp.float32)

        base = s * rpt

        @pl.loop(0, nfull)
        def _za(k):
            pltpu.sync_copy(buf_a, acc_sh.at[pl.ds(base + k * chunk, chunk)])

        if rem:
            pltpu.sync_copy(buf_a.at[pl.ds(0, rem)],
                            acc_sh.at[pl.ds(base + nfull * chunk, rem)])

        plsc.subcore_barrier()

        def _gather(j, buf, sem):
            return pltpu.make_async_copy(
                g_hbm.at[src_v.at[pl.ds(j * chunk, chunk)]], buf, sem)

        def _scatter(j, buf):
            pltpu.sync_copy(buf, acc_sh.at[dst_v.at[j]], add=True)

        _gather(0, buf_a, sem_a).start()

        @pl.loop(0, nch // 2)
        def _pair(k):
            j = 2 * k

            @pl.when(j + 1 < nch)
            def _():
                _gather(j + 1, buf_b, sem_b).start()

            _gather(j, buf_a, sem_a).wait()
            _scatter(j, buf_a)

            @pl.when(j + 2 < nch)
            def _():
                _gather(j + 2, buf_a, sem_a).start()

            @pl.when(j + 1 < nch)
            def _():
                _gather(j + 1, buf_b, sem_b).wait()
                _scatter(j + 1, buf_b)

        if nch % 2:
            _gather(nch - 1, buf_a, sem_a).wait()
            _scatter(nch - 1, buf_a)

        plsc.subcore_barrier()

        @pl.loop(0, nfull)
        def _out(k):
            pltpu.sync_copy(acc_sh.at[pl.ds(base + k * chunk, chunk)], buf_a)
            pltpu.sync_copy(buf_a, acc_hbm.at[c, pl.ds(base + k * chunk, chunk)])

        if rem:
            pltpu.sync_copy(acc_sh.at[pl.ds(base + nfull * chunk, rem)],
                            buf_a.at[pl.ds(0, rem)])
            pltpu.sync_copy(buf_a.at[pl.ds(0, rem)],
                            acc_hbm.at[c, pl.ds(base + nfull * chunk, rem)])

    return scat_kernel(g, src2, dst3)


def _tca_body(x_ref, w_ref, degp_ref, g_ref):
    deg = jnp.sum(degp_ref[...], axis=0) + 1.0
    dinv = lax.rsqrt(deg)
    h = jnp.dot(x_ref[...], w_ref[...], preferred_element_type=jnp.float32)
    g_ref[...] = h * dinv[:, None]


def _tc_transform(x, W, degp):
    n, d_in = x.shape
    d = W.shape[1]
    br = 512
    return pl.pallas_call(
        _tca_body,
        grid=(pl.cdiv(n, br),),
        in_specs=[
            pl.BlockSpec((br, d_in), lambda i: (i, 0)),
            pl.BlockSpec((d_in, d), lambda i: (0, 0)),
            pl.BlockSpec((NC, br), lambda i: (0, i)),
        ],
        out_specs=pl.BlockSpec((br, d), lambda i: (i, 0)),
        out_shape=jax.ShapeDtypeStruct((n, d), jnp.float32),
    )(x, W, degp)


def _tcb_body(accp_ref, g_ref, degp_ref, b_ref, o_ref):
    deg = jnp.sum(degp_ref[...], axis=0) + 1.0
    dinv = lax.rsqrt(deg)
    z = (accp_ref[0] + accp_ref[1] + g_ref[...]) * dinv[:, None] + b_ref[...]
    m = jnp.max(z, axis=1, keepdims=True)
    ez = jnp.exp(z - m)
    o_ref[...] = z - m - jnp.log(jnp.sum(ez, axis=1, keepdims=True))


def _tc_finalize(accp, g, degp, b):
    n, d = g.shape
    br = 512
    return pl.pallas_call(
        _tcb_body,
        grid=(pl.cdiv(n, br),),
        in_specs=[
            pl.BlockSpec((NC, br, d), lambda i: (0, i, 0)),
            pl.BlockSpec((br, d), lambda i: (i, 0)),
            pl.BlockSpec((NC, br), lambda i: (0, i)),
            pl.BlockSpec((1, d), lambda i: (0, 0)),
        ],
        out_specs=pl.BlockSpec((br, d), lambda i: (i, 0)),
        out_shape=jax.ShapeDtypeStruct((n, d), jnp.float32),
    )(accp, g, degp, b.reshape(1, d))


def kernel(x, edge_index, W, b):
    n, d_in = x.shape
    d = W.shape[1]
    e = edge_index.shape[1]
    src = edge_index[0]
    dst = edge_index[1]

    chunk_d, nch_d, pad_d = _plan_edges(e)
    nrows_d = _round_up(n, 256)
    dst_d = dst
    if pad_d:
        ar = jnp.arange(pad_d, dtype=jnp.int32)
        dst_d = jnp.concatenate([dst, n + (ar % 64)])
        nrows_d = _round_up(n + 64, 256)

    chunk_s = 64
    nch_s = -(-e // (NW * chunk_s))
    pad_s = NW * chunk_s * nch_s - e
    n_junk = 16
    nrows_s = _round_up(n + n_junk, 128)
    src_s, dst_s = src, dst
    if pad_s:
        ar = jnp.arange(pad_s, dtype=jnp.int32)
        src_s = jnp.concatenate([src, ar % n])
        dst_s = jnp.concatenate([dst, n + (ar % n_junk)])

    degp = _sc_degree(dst_d.reshape(NW, nch_d, chunk_d), nrows_d)
    g = _tc_transform(x, W, degp)
    accp = _sc_scatter(g, src_s.reshape(NW, nch_s * chunk_s),
                       dst_s.reshape(NW, nch_s, chunk_s), nrows_s)
    out = _tc_finalize(accp, g, degp, b)
    return out


# speedup vs baseline: 1.4709x; 1.0659x over previous
import functools

import jax
import jax.numpy as jnp
from jax import lax
from jax.experimental import pallas as pl
from jax.experimental.pallas import tpu as pltpu
from jax.experimental.pallas import tpu_sc as plsc

NC = 2
NS = 16
NW = NC * NS


def _round_up(a, m):
    return -(-a // m) * m


def _plan_edges(e, max_chunk=128):
    for chunk in range(max_chunk, 0, -8):
        if e % (NW * chunk) == 0:
            return chunk, e // (NW * chunk), 0
    chunk = max_chunk
    nch = -(-e // (NW * chunk))
    return chunk, nch, NW * chunk * nch - e


def _sc_degree(dst3, nrows):
    nw, nch, chunk = dst3.shape
    rpt = nrows // NS

    @functools.partial(
        pl.kernel,
        out_type=jax.ShapeDtypeStruct((NC, nrows), jnp.float32),
        mesh=plsc.VectorSubcoreMesh(core_axis_name="c", subcore_axis_name="s"),
        scratch_types=[
            pltpu.VMEM((nch, chunk), jnp.int32),
            pltpu.VMEM((chunk,), jnp.float32),
            pltpu.VMEM((rpt,), jnp.float32),
            pltpu.VMEM_SHARED((nrows,), jnp.float32),
            pltpu.SemaphoreType.DMA,
        ],
    )
    def deg_kernel(dst_hbm, degp_hbm, idx_v, ones_v, zb_v, deg_sh, sem):
        c = lax.axis_index("c")
        s = lax.axis_index("s")
        w = c * NS + s
        pltpu.sync_copy(dst_hbm.at[w], idx_v)

        @pl.loop(0, chunk // 16)
        def _ones(i):
            ones_v[pl.ds(i * 16, 16)] = jnp.ones((16,), jnp.float32)

        @pl.loop(0, rpt // 16)
        def _zb(i):
            zb_v[pl.ds(i * 16, 16)] = jnp.zeros((16,), jnp.float32)

        pltpu.sync_copy(zb_v, deg_sh.at[pl.ds(s * rpt, rpt)])
        plsc.subcore_barrier()

        grp = 5 if nch % 5 == 0 else 1

        @pl.loop(0, nch // grp)
        def _scat(gi):
            for u in range(grp):
                dsc = pltpu.make_async_copy(
                    ones_v, deg_sh.at[idx_v.at[gi * grp + u]], sem)
                dsc.start(add=True)
            for u in range(grp):
                pltpu.make_async_copy(
                    ones_v, deg_sh.at[idx_v.at[gi * grp + u]], sem).wait()

        plsc.subcore_barrier()
        pltpu.sync_copy(deg_sh.at[pl.ds(s * rpt, rpt)], zb_v)
        pltpu.sync_copy(zb_v, degp_hbm.at[c, pl.ds(s * rpt, rpt)])

    return deg_kernel(dst3)


def _sc_scatter(g, src2, dst3, nrows):
    nw, nch, chunk = dst3.shape
    d = g.shape[1]
    rpt = nrows // NS
    nfull = rpt // chunk
    rem = rpt % chunk

    @functools.partial(
        pl.kernel,
        out_type=jax.ShapeDtypeStruct((NC, nrows, d), jnp.float32),
        mesh=plsc.VectorSubcoreMesh(core_axis_name="c", subcore_axis_name="s"),
        scratch_types=[
            pltpu.VMEM((nch * chunk,), jnp.int32),
            pltpu.VMEM((nch, chunk), jnp.int32),
            pltpu.VMEM((chunk, d), jnp.float32),
            pltpu.VMEM((chunk, d), jnp.float32),
            pltpu.VMEM_SHARED((nrows, d), jnp.float32),
            pltpu.SemaphoreType.DMA,
            pltpu.SemaphoreType.DMA,
        ],
    )
    def scat_kernel(g_hbm, src_hbm, dst_hbm, acc_hbm,
                    src_v, dst_v, buf_a, buf_b, acc_sh, sem_a, sem_b):
        c = lax.axis_index("c")
        s = lax.axis_index("s")
        w = c * NS + s
        pltpu.sync_copy(src_hbm.at[w], src_v)
        pltpu.sync_copy(dst_hbm.at[w], dst_v)

        @pl.loop(0, chunk)
        def _zr(r):
            @pl.loop(0, d // 16)
            def _zc(i):
                buf_a[r, pl.ds(i * 16, 16)] = jnp.zeros((16,), jnp.float32)

        base = s * rpt

        @pl.loop(0, nfull)
        def _za(k):
            pltpu.sync_copy(buf_a, acc_sh.at[pl.ds(base + k * chunk, chunk)])

        if rem:
            pltpu.sync_copy(buf_a.at[pl.ds(0, rem)],
                            acc_sh.at[pl.ds(base + nfull * chunk, rem)])

        plsc.subcore_barrier()

        def _gather(j, buf, sem):
            return pltpu.make_async_copy(
                g_hbm.at[src_v.at[pl.ds(j * chunk, chunk)]], buf, sem)

        def _scatter(j, buf):
            pltpu.sync_copy(buf, acc_sh.at[dst_v.at[j]], add=True)

        _gather(0, buf_a, sem_a).start()

        @pl.loop(0, nch // 2)
        def _pair(k):
            j = 2 * k

            @pl.when(j + 1 < nch)
            def _():
                _gather(j + 1, buf_b, sem_b).start()

            _gather(j, buf_a, sem_a).wait()
            _scatter(j, buf_a)

            @pl.when(j + 2 < nch)
            def _():
                _gather(j + 2, buf_a, sem_a).start()

            @pl.when(j + 1 < nch)
            def _():
                _gather(j + 1, buf_b, sem_b).wait()
                _scatter(j + 1, buf_b)

        if nch % 2:
            _gather(nch - 1, buf_a, sem_a).wait()
            _scatter(nch - 1, buf_a)

        plsc.subcore_barrier()

        @pl.loop(0, nfull)
        def _out(k):
            pltpu.sync_copy(acc_sh.at[pl.ds(base + k * chunk, chunk)], buf_a)
            pltpu.sync_copy(buf_a, acc_hbm.at[c, pl.ds(base + k * chunk, chunk)])

        if rem:
            pltpu.sync_copy(acc_sh.at[pl.ds(base + nfull * chunk, rem)],
                            buf_a.at[pl.ds(0, rem)])
            pltpu.sync_copy(buf_a.at[pl.ds(0, rem)],
                            acc_hbm.at[c, pl.ds(base + nfull * chunk, rem)])

    return scat_kernel(g, src2, dst3)


def _mm_body(x_ref, w_ref, h_ref):
    h_ref[...] = jnp.dot(x_ref[...], w_ref[...],
                         preferred_element_type=jnp.float32)


def _tc_matmul(x, W):
    n, d_in = x.shape
    d = W.shape[1]
    br = 512
    return pl.pallas_call(
        _mm_body,
        grid=(pl.cdiv(n, br),),
        in_specs=[
            pl.BlockSpec((br, d_in), lambda i: (i, 0)),
            pl.BlockSpec((d_in, d), lambda i: (0, 0)),
        ],
        out_specs=pl.BlockSpec((br, d), lambda i: (i, 0)),
        out_shape=jax.ShapeDtypeStruct((n, d), jnp.float32),
    )(x, W)


def _scale_body(h_ref, degp_ref, g_ref):
    deg = jnp.sum(degp_ref[...], axis=0) + 1.0
    dinv = lax.rsqrt(deg)
    g_ref[...] = h_ref[...] * dinv[:, None]


def _tc_scale(h, degp):
    n, d = h.shape
    br = 512
    return pl.pallas_call(
        _scale_body,
        grid=(pl.cdiv(n, br),),
        in_specs=[
            pl.BlockSpec((br, d), lambda i: (i, 0)),
            pl.BlockSpec((NC, br), lambda i: (0, i)),
        ],
        out_specs=pl.BlockSpec((br, d), lambda i: (i, 0)),
        out_shape=jax.ShapeDtypeStruct((n, d), jnp.float32),
    )(h, degp)


def _tcb_body(accp_ref, g_ref, degp_ref, b_ref, o_ref):
    deg = jnp.sum(degp_ref[...], axis=0) + 1.0
    dinv = lax.rsqrt(deg)
    z = (accp_ref[0] + accp_ref[1] + g_ref[...]) * dinv[:, None] + b_ref[...]
    m = jnp.max(z, axis=1, keepdims=True)
    ez = jnp.exp(z - m)
    o_ref[...] = z - m - jnp.log(jnp.sum(ez, axis=1, keepdims=True))


def _tc_finalize(accp, g, degp, b):
    n, d = g.shape
    br = 512
    return pl.pallas_call(
        _tcb_body,
        grid=(pl.cdiv(n, br),),
        in_specs=[
            pl.BlockSpec((NC, br, d), lambda i: (0, i, 0)),
            pl.BlockSpec((br, d), lambda i: (i, 0)),
            pl.BlockSpec((NC, br), lambda i: (0, i)),
            pl.BlockSpec((1, d), lambda i: (0, 0)),
        ],
        out_specs=pl.BlockSpec((br, d), lambda i: (i, 0)),
        out_shape=jax.ShapeDtypeStruct((n, d), jnp.float32),
    )(accp, g, degp, b.reshape(1, d))


def kernel(x, edge_index, W, b):
    n, d_in = x.shape
    d = W.shape[1]
    e = edge_index.shape[1]
    src = edge_index[0]
    dst = edge_index[1]

    chunk_d, nch_d, pad_d = _plan_edges(e)
    nrows_d = _round_up(n, 256)
    dst_d = dst
    if pad_d:
        ar = jnp.arange(pad_d, dtype=jnp.int32)
        dst_d = jnp.concatenate([dst, n + (ar % 64)])
        nrows_d = _round_up(n + 64, 256)

    chunk_s = 64
    nch_s = -(-e // (NW * chunk_s))
    pad_s = NW * chunk_s * nch_s - e
    n_junk = 16
    nrows_s = _round_up(n + n_junk, 128)
    src_s, dst_s = src, dst
    if pad_s:
        ar = jnp.arange(pad_s, dtype=jnp.int32)
        src_s = jnp.concatenate([src, ar % n])
        dst_s = jnp.concatenate([dst, n + (ar % n_junk)])

    degp = _sc_degree(dst_d.reshape(NW, nch_d, chunk_d), nrows_d)
    h = _tc_matmul(x, W)
    g = _tc_scale(h, degp)
    accp = _sc_scatter(g, src_s.reshape(NW, nch_s * chunk_s),
                       dst_s.reshape(NW, nch_s, chunk_s), nrows_s)
    out = _tc_finalize(accp, g, degp, b)
    return out


# speedup vs baseline: 1.4717x; 1.0005x over previous
import functools

import jax
import jax.numpy as jnp
from jax import lax
from jax.experimental import pallas as pl
from jax.experimental.pallas import tpu as pltpu
from jax.experimental.pallas import tpu_sc as plsc

NC = 2
NS = 16
NW = NC * NS


def _round_up(a, m):
    return -(-a // m) * m


def _plan_edges(e, max_chunk=128):
    for chunk in range(max_chunk, 0, -8):
        if e % (NW * chunk) == 0:
            return chunk, e // (NW * chunk), 0
    chunk = max_chunk
    nch = -(-e // (NW * chunk))
    return chunk, nch, NW * chunk * nch - e


def _sc_degree(dst3, nrows):
    nw, nch, chunk = dst3.shape
    rpt = nrows // NS

    @functools.partial(
        pl.kernel,
        out_type=jax.ShapeDtypeStruct((NC, nrows), jnp.float32),
        mesh=plsc.VectorSubcoreMesh(core_axis_name="c", subcore_axis_name="s"),
        scratch_types=[
            pltpu.VMEM((nch, chunk), jnp.int32),
            pltpu.VMEM((chunk,), jnp.float32),
            pltpu.VMEM((rpt,), jnp.float32),
            pltpu.VMEM_SHARED((nrows,), jnp.float32),
            pltpu.SemaphoreType.DMA,
        ],
    )
    def deg_kernel(dst_hbm, degp_hbm, idx_v, ones_v, zb_v, deg_sh, sem):
        c = lax.axis_index("c")
        s = lax.axis_index("s")
        w = c * NS + s
        pltpu.sync_copy(dst_hbm.at[w], idx_v)

        @pl.loop(0, chunk // 16)
        def _ones(i):
            ones_v[pl.ds(i * 16, 16)] = jnp.ones((16,), jnp.float32)

        @pl.loop(0, rpt // 16)
        def _zb(i):
            zb_v[pl.ds(i * 16, 16)] = jnp.zeros((16,), jnp.float32)

        pltpu.sync_copy(zb_v, deg_sh.at[pl.ds(s * rpt, rpt)])
        plsc.subcore_barrier()

        grp = 5 if nch % 5 == 0 else 1

        @pl.loop(0, nch // grp)
        def _scat(gi):
            for u in range(grp):
                dsc = pltpu.make_async_copy(
                    ones_v, deg_sh.at[idx_v.at[gi * grp + u]], sem)
                dsc.start(add=True)
            for u in range(grp):
                pltpu.make_async_copy(
                    ones_v, deg_sh.at[idx_v.at[gi * grp + u]], sem).wait()

        plsc.subcore_barrier()
        pltpu.sync_copy(deg_sh.at[pl.ds(s * rpt, rpt)],
                        degp_hbm.at[c, pl.ds(s * rpt, rpt)])

    return deg_kernel(dst3)


def _sc_scatter(g, src2, dst3, nrows):
    nw, nch, chunk = dst3.shape
    d = g.shape[1]
    adt = g.dtype
    rpt = nrows // NS
    nfull = rpt // chunk
    rem = rpt % chunk
    lanes = 32 if adt == jnp.bfloat16 else 16

    @functools.partial(
        pl.kernel,
        out_type=jax.ShapeDtypeStruct((NC, nrows, d), adt),
        mesh=plsc.VectorSubcoreMesh(core_axis_name="c", subcore_axis_name="s"),
        scratch_types=[
            pltpu.VMEM((nch * chunk,), jnp.int32),
            pltpu.VMEM((nch, chunk), jnp.int32),
            pltpu.VMEM((chunk, d), adt),
            pltpu.VMEM((chunk, d), adt),
            pltpu.VMEM_SHARED((nrows, d), adt),
            pltpu.SemaphoreType.DMA,
            pltpu.SemaphoreType.DMA,
        ],
    )
    def scat_kernel(g_hbm, src_hbm, dst_hbm, acc_hbm,
                    src_v, dst_v, buf_a, buf_b, acc_sh, sem_a, sem_b):
        c = lax.axis_index("c")
        s = lax.axis_index("s")
        w = c * NS + s
        pltpu.sync_copy(src_hbm.at[w], src_v)
        pltpu.sync_copy(dst_hbm.at[w], dst_v)

        if adt == jnp.bfloat16:
            @pl.loop(0, chunk // 2)
            def _zr(r2):
                rr = pl.multiple_of(r2 * 2, 2)

                @pl.loop(0, d // 16)
                def _zc(i):
                    buf_a[pl.ds(rr, 2), pl.ds(i * 16, 16)] = jnp.zeros(
                        (2, 16), adt)
        else:
            @pl.loop(0, chunk)
            def _zr(r):
                @pl.loop(0, d // lanes)
                def _zc(i):
                    buf_a[r, pl.ds(i * lanes, lanes)] = jnp.zeros((lanes,), adt)

        base = s * rpt

        @pl.loop(0, nfull)
        def _za(k):
            pltpu.sync_copy(buf_a, acc_sh.at[pl.ds(base + k * chunk, chunk)])

        if rem:
            pltpu.sync_copy(buf_a.at[pl.ds(0, rem)],
                            acc_sh.at[pl.ds(base + nfull * chunk, rem)])

        plsc.subcore_barrier()

        def _gather(j, buf, sem):
            return pltpu.make_async_copy(
                g_hbm.at[src_v.at[pl.ds(j * chunk, chunk)]], buf, sem)

        def _scatter(j, buf):
            pltpu.sync_copy(buf, acc_sh.at[dst_v.at[j]], add=True)

        _gather(0, buf_a, sem_a).start()

        @pl.loop(0, nch // 2)
        def _pair(k):
            j = 2 * k

            @pl.when(j + 1 < nch)
            def _():
                _gather(j + 1, buf_b, sem_b).start()

            _gather(j, buf_a, sem_a).wait()
            _scatter(j, buf_a)

            @pl.when(j + 2 < nch)
            def _():
                _gather(j + 2, buf_a, sem_a).start()

            @pl.when(j + 1 < nch)
            def _():
                _gather(j + 1, buf_b, sem_b).wait()
                _scatter(j + 1, buf_b)

        if nch % 2:
            _gather(nch - 1, buf_a, sem_a).wait()
            _scatter(nch - 1, buf_a)

        plsc.subcore_barrier()

        pltpu.sync_copy(acc_sh.at[pl.ds(base, rpt)],
                        acc_hbm.at[c, pl.ds(base, rpt)])

    return scat_kernel(g, src2, dst3)


def _mm_body(x_ref, w_ref, h_ref):
    h_ref[...] = jnp.dot(x_ref[...], w_ref[...],
                         preferred_element_type=jnp.float32)


def _tc_matmul(x, W):
    n, d_in = x.shape
    d = W.shape[1]
    br = 512
    return pl.pallas_call(
        _mm_body,
        grid=(pl.cdiv(n, br),),
        in_specs=[
            pl.BlockSpec((br, d_in), lambda i: (i, 0)),
            pl.BlockSpec((d_in, d), lambda i: (0, 0)),
        ],
        out_specs=pl.BlockSpec((br, d), lambda i: (i, 0)),
        out_shape=jax.ShapeDtypeStruct((n, d), jnp.float32),
    )(x, W)


def _scale_body(h_ref, degp_ref, g_ref):
    deg = jnp.sum(degp_ref[...], axis=0) + 1.0
    dinv = lax.rsqrt(deg)
    g_ref[...] = (h_ref[...] * dinv[:, None]).astype(g_ref.dtype)


def _tc_scale(h, degp, dtype):
    n, d = h.shape
    br = 512
    return pl.pallas_call(
        _scale_body,
        grid=(pl.cdiv(n, br),),
        in_specs=[
            pl.BlockSpec((br, d), lambda i: (i, 0)),
            pl.BlockSpec((NC, br), lambda i: (0, i)),
        ],
        out_specs=pl.BlockSpec((br, d), lambda i: (i, 0)),
        out_shape=jax.ShapeDtypeStruct((n, d), dtype),
    )(h, degp)


def _tcb_body(accp_ref, g_ref, degp_ref, b_ref, o_ref):
    deg = jnp.sum(degp_ref[...], axis=0) + 1.0
    dinv = lax.rsqrt(deg)
    acc = (accp_ref[0].astype(jnp.float32) + accp_ref[1].astype(jnp.float32)
           + g_ref[...].astype(jnp.float32))
    z = acc * dinv[:, None] + b_ref[...]
    m = jnp.max(z, axis=1, keepdims=True)
    ez = jnp.exp(z - m)
    o_ref[...] = z - m - jnp.log(jnp.sum(ez, axis=1, keepdims=True))


def _tc_finalize(accp, g, degp, b):
    n, d = g.shape
    br = 512
    return pl.pallas_call(
        _tcb_body,
        grid=(pl.cdiv(n, br),),
        in_specs=[
            pl.BlockSpec((NC, br, d), lambda i: (0, i, 0)),
            pl.BlockSpec((br, d), lambda i: (i, 0)),
            pl.BlockSpec((NC, br), lambda i: (0, i)),
            pl.BlockSpec((1, d), lambda i: (0, 0)),
        ],
        out_specs=pl.BlockSpec((br, d), lambda i: (i, 0)),
        out_shape=jax.ShapeDtypeStruct((n, d), jnp.float32),
    )(accp, g, degp, b.reshape(1, d))


def kernel(x, edge_index, W, b):
    n, d_in = x.shape
    d = W.shape[1]
    e = edge_index.shape[1]
    src = edge_index[0]
    dst = edge_index[1]

    chunk_d, nch_d, pad_d = _plan_edges(e)
    nrows_d = _round_up(n, 256)
    dst_d = dst
    if pad_d:
        ar = jnp.arange(pad_d, dtype=jnp.int32)
        dst_d = jnp.concatenate([dst, n + (ar % 64)])
        nrows_d = _round_up(n + 64, 256)

    chunk_s = 96
    nch_s = -(-e // (NW * chunk_s))
    pad_s = NW * chunk_s * nch_s - e
    n_junk = 16
    nrows_s = _round_up(n + n_junk, 128)
    src_s, dst_s = src, dst
    if pad_s:
        ar = jnp.arange(pad_s, dtype=jnp.int32)
        src_s = jnp.concatenate([src, ar % n])
        dst_s = jnp.concatenate([dst, n + (ar % n_junk)])

    degp = _sc_degree(dst_d.reshape(NW, nch_d, chunk_d), nrows_d)
    h = _tc_matmul(x, W)
    g = _tc_scale(h, degp, jnp.float32)
    accp = _sc_scatter(g, src_s.reshape(NW, nch_s * chunk_s),
                       dst_s.reshape(NW, nch_s, chunk_s), nrows_s)
    out = _tc_finalize(accp, g, degp, b)
    return out


# speedup vs baseline: 1.5384x; 1.0453x over previous
import functools

import jax
import jax.numpy as jnp
from jax import lax
from jax.experimental import pallas as pl
from jax.experimental.pallas import tpu as pltpu
from jax.experimental.pallas import tpu_sc as plsc

NC = 2
NS = 16
NW = NC * NS


def _round_up(a, m):
    return -(-a // m) * m


def _plan_edges(e, max_chunk=128):
    for chunk in range(max_chunk, 0, -8):
        if e % (NW * chunk) == 0:
            return chunk, e // (NW * chunk), 0
    chunk = max_chunk
    nch = -(-e // (NW * chunk))
    return chunk, nch, NW * chunk * nch - e


def _sc_degree(dst3, nrows):
    nw, nch, chunk = dst3.shape
    rpt = nrows // NS

    @functools.partial(
        pl.kernel,
        out_type=jax.ShapeDtypeStruct((NC, nrows), jnp.float32),
        mesh=plsc.VectorSubcoreMesh(core_axis_name="c", subcore_axis_name="s"),
        scratch_types=[
            pltpu.VMEM((nch, chunk), jnp.int32),
            pltpu.VMEM((chunk,), jnp.float32),
            pltpu.VMEM((rpt,), jnp.float32),
            pltpu.VMEM_SHARED((nrows,), jnp.float32),
            pltpu.SemaphoreType.DMA,
        ],
    )
    def deg_kernel(dst_hbm, degp_hbm, idx_v, ones_v, zb_v, deg_sh, sem):
        c = lax.axis_index("c")
        s = lax.axis_index("s")
        w = c * NS + s
        pltpu.sync_copy(dst_hbm.at[w], idx_v)

        @pl.loop(0, chunk // 16)
        def _ones(i):
            ones_v[pl.ds(i * 16, 16)] = jnp.ones((16,), jnp.float32)

        @pl.loop(0, rpt // 16)
        def _zb(i):
            zb_v[pl.ds(i * 16, 16)] = jnp.zeros((16,), jnp.float32)

        pltpu.sync_copy(zb_v, deg_sh.at[pl.ds(s * rpt, rpt)])
        plsc.subcore_barrier()

        grp = 5 if nch % 5 == 0 else 1

        @pl.loop(0, nch // grp)
        def _scat(gi):
            for u in range(grp):
                dsc = pltpu.make_async_copy(
                    ones_v, deg_sh.at[idx_v.at[gi * grp + u]], sem)
                dsc.start(add=True)
            for u in range(grp):
                pltpu.make_async_copy(
                    ones_v, deg_sh.at[idx_v.at[gi * grp + u]], sem).wait()

        plsc.subcore_barrier()
        pltpu.sync_copy(deg_sh.at[pl.ds(s * rpt, rpt)],
                        degp_hbm.at[c, pl.ds(s * rpt, rpt)])

    return deg_kernel(dst3)


def _sc_scatter(g, src2, dst3, nrows):
    nw, nch, chunk = dst3.shape
    d = g.shape[1]
    adt = g.dtype
    rpt = nrows // NS
    nfull = rpt // chunk
    rem = rpt % chunk
    lanes = 32 if adt == jnp.bfloat16 else 16

    @functools.partial(
        pl.kernel,
        out_type=jax.ShapeDtypeStruct((NC, nrows, d), adt),
        mesh=plsc.VectorSubcoreMesh(core_axis_name="c", subcore_axis_name="s"),
        scratch_types=[
            pltpu.VMEM((nch * chunk,), jnp.int32),
            pltpu.VMEM((nch, chunk), jnp.int32),
            pltpu.VMEM((chunk, d), adt),
            pltpu.VMEM((chunk, d), adt),
            pltpu.VMEM_SHARED((nrows, d), adt),
            pltpu.SemaphoreType.DMA,
            pltpu.SemaphoreType.DMA,
        ],
    )
    def scat_kernel(g_hbm, src_hbm, dst_hbm, acc_hbm,
                    src_v, dst_v, buf_a, buf_b, acc_sh, sem_a, sem_b):
        c = lax.axis_index("c")
        s = lax.axis_index("s")
        w = c * NS + s
        pltpu.sync_copy(src_hbm.at[w], src_v)
        pltpu.sync_copy(dst_hbm.at[w], dst_v)

        base = s * rpt

        @pl.when(c == 0)
        def _seed():
            pltpu.sync_copy(g_hbm.at[pl.ds(base, rpt)],
                            acc_sh.at[pl.ds(base, rpt)])

        @pl.when(c != 0)
        def _zfill():
            @pl.loop(0, chunk)
            def _zr(r):
                @pl.loop(0, d // lanes)
                def _zc(i):
                    buf_a[r, pl.ds(i * lanes, lanes)] = jnp.zeros((lanes,), adt)

            @pl.loop(0, nfull)
            def _za(k):
                pltpu.sync_copy(buf_a, acc_sh.at[pl.ds(base + k * chunk, chunk)])

            if rem:
                pltpu.sync_copy(buf_a.at[pl.ds(0, rem)],
                                acc_sh.at[pl.ds(base + nfull * chunk, rem)])

        plsc.subcore_barrier()

        def _gather(j, buf, sem):
            return pltpu.make_async_copy(
                g_hbm.at[src_v.at[pl.ds(j * chunk, chunk)]], buf, sem)

        def _scatter(j, buf):
            pltpu.sync_copy(buf, acc_sh.at[dst_v.at[j]], add=True)

        _gather(0, buf_a, sem_a).start()

        @pl.loop(0, nch // 2)
        def _pair(k):
            j = 2 * k

            @pl.when(j + 1 < nch)
            def _():
                _gather(j + 1, buf_b, sem_b).start()

            _gather(j, buf_a, sem_a).wait()
            _scatter(j, buf_a)

            @pl.when(j + 2 < nch)
            def _():
                _gather(j + 2, buf_a, sem_a).start()

            @pl.when(j + 1 < nch)
            def _():
                _gather(j + 1, buf_b, sem_b).wait()
                _scatter(j + 1, buf_b)

        if nch % 2:
            _gather(nch - 1, buf_a, sem_a).wait()
            _scatter(nch - 1, buf_a)

        plsc.subcore_barrier()

        pltpu.sync_copy(acc_sh.at[pl.ds(base, rpt)],
                        acc_hbm.at[c, pl.ds(base, rpt)])

    return scat_kernel(g, src2, dst3)


def _mm_body(x_ref, w_ref, degp_ref, g_ref):
    deg = jnp.sum(degp_ref[...], axis=0) + 1.0
    dinv = lax.rsqrt(deg)
    xs = x_ref[...] * dinv[:, None]
    g_ref[...] = jnp.dot(xs, w_ref[...], preferred_element_type=jnp.float32)


def _tc_matmul(x, W, degp, nr):
    n, d_in = x.shape
    d = W.shape[1]
    br = 512
    return pl.pallas_call(
        _mm_body,
        grid=(pl.cdiv(nr, br),),
        in_specs=[
            pl.BlockSpec((br, d_in), lambda i: (i, 0)),
            pl.BlockSpec((d_in, d), lambda i: (0, 0)),
            pl.BlockSpec((NC, br), lambda i: (0, i)),
        ],
        out_specs=pl.BlockSpec((br, d), lambda i: (i, 0)),
        out_shape=jax.ShapeDtypeStruct((nr, d), jnp.float32),
    )(x, W, degp)


def _tcb_body(accp_ref, degp_ref, b_ref, o_ref):
    deg = jnp.sum(degp_ref[...], axis=0) + 1.0
    dinv = lax.rsqrt(deg)
    acc = accp_ref[0].astype(jnp.float32) + accp_ref[1].astype(jnp.float32)
    z = acc * dinv[:, None] + b_ref[...]
    m = jnp.max(z, axis=1, keepdims=True)
    ez = jnp.exp(z - m)
    o_ref[...] = z - m - jnp.log(jnp.sum(ez, axis=1, keepdims=True))


def _tc_finalize(accp, degp, b, n):
    d = accp.shape[2]
    br = 512
    return pl.pallas_call(
        _tcb_body,
        grid=(pl.cdiv(n, br),),
        in_specs=[
            pl.BlockSpec((NC, br, d), lambda i: (0, i, 0)),
            pl.BlockSpec((NC, br), lambda i: (0, i)),
            pl.BlockSpec((1, d), lambda i: (0, 0)),
        ],
        out_specs=pl.BlockSpec((br, d), lambda i: (i, 0)),
        out_shape=jax.ShapeDtypeStruct((n, d), jnp.float32),
    )(accp, degp, b.reshape(1, d))


def kernel(x, edge_index, W, b):
    n, d_in = x.shape
    d = W.shape[1]
    e = edge_index.shape[1]
    src = edge_index[0]
    dst = edge_index[1]

    chunk_d, nch_d, pad_d = _plan_edges(e)
    nrows_d = _round_up(n, 256)
    dst_d = dst
    if pad_d:
        ar = jnp.arange(pad_d, dtype=jnp.int32)
        dst_d = jnp.concatenate([dst, n + (ar % 64)])
        nrows_d = _round_up(n + 64, 256)

    chunk_s = 96
    nch_s = -(-e // (NW * chunk_s))
    pad_s = NW * chunk_s * nch_s - e
    n_junk = 16
    nrows_s = _round_up(n + n_junk, 128)
    src_s, dst_s = src, dst
    if pad_s:
        ar = jnp.arange(pad_s, dtype=jnp.int32)
        src_s = jnp.concatenate([src, ar % n])
        dst_s = jnp.concatenate([dst, n + (ar % n_junk)])

    degp = _sc_degree(dst_d.reshape(NW, nch_d, chunk_d), nrows_d)
    g = _tc_matmul(x, W, degp, nrows_s)
    accp = _sc_scatter(g, src_s.reshape(NW, nch_s * chunk_s),
                       dst_s.reshape(NW, nch_s, chunk_s), nrows_s)
    out = _tc_finalize(accp, degp, b, n)
    return out


# speedup vs baseline: 1.5451x; 1.0044x over previous
import functools

import jax
import jax.numpy as jnp
from jax import lax
from jax.experimental import pallas as pl
from jax.experimental.pallas import tpu as pltpu
from jax.experimental.pallas import tpu_sc as plsc

NC = 2
NS = 16
NW = NC * NS


def _round_up(a, m):
    return -(-a // m) * m


def _plan_edges(e, max_chunk=128):
    for chunk in range(max_chunk, 0, -8):
        if e % (NW * chunk) == 0:
            return chunk, e // (NW * chunk), 0
    chunk = max_chunk
    nch = -(-e // (NW * chunk))
    return chunk, nch, NW * chunk * nch - e


def _sc_degree(dst3, nrows):
    nw, nch, chunk = dst3.shape
    rpt = nrows // NS

    @functools.partial(
        pl.kernel,
        out_type=jax.ShapeDtypeStruct((NC, nrows), jnp.float32),
        mesh=plsc.VectorSubcoreMesh(core_axis_name="c", subcore_axis_name="s"),
        scratch_types=[
            pltpu.VMEM((nch, chunk), jnp.int32),
            pltpu.VMEM((chunk,), jnp.float32),
            pltpu.VMEM((rpt,), jnp.float32),
            pltpu.VMEM_SHARED((nrows,), jnp.float32),
            pltpu.SemaphoreType.DMA,
        ],
    )
    def deg_kernel(dst_hbm, degp_hbm, idx_v, ones_v, zb_v, deg_sh, sem):
        c = lax.axis_index("c")
        s = lax.axis_index("s")
        w = c * NS + s
        pltpu.sync_copy(dst_hbm.at[w], idx_v)

        @pl.loop(0, chunk // 16)
        def _ones(i):
            ones_v[pl.ds(i * 16, 16)] = jnp.ones((16,), jnp.float32)

        @pl.loop(0, rpt // 16)
        def _zb(i):
            zb_v[pl.ds(i * 16, 16)] = jnp.zeros((16,), jnp.float32)

        pltpu.sync_copy(zb_v, deg_sh.at[pl.ds(s * rpt, rpt)])
        plsc.subcore_barrier()

        grp = 5 if nch % 5 == 0 else 1

        @pl.loop(0, nch // grp)
        def _scat(gi):
            for u in range(grp):
                dsc = pltpu.make_async_copy(
                    ones_v, deg_sh.at[idx_v.at[gi * grp + u]], sem)
                dsc.start(add=True)
            for u in range(grp):
                pltpu.make_async_copy(
                    ones_v, deg_sh.at[idx_v.at[gi * grp + u]], sem).wait()

        plsc.subcore_barrier()
        pltpu.sync_copy(deg_sh.at[pl.ds(s * rpt, rpt)],
                        degp_hbm.at[c, pl.ds(s * rpt, rpt)])

    return deg_kernel(dst3)


def _sc_scatter(g, src2, dst3, nrows):
    nw, nch, chunk = dst3.shape
    d = g.shape[1]
    adt = g.dtype
    rpt = nrows // NS
    nfull = rpt // chunk
    rem = rpt % chunk
    lanes = 32 if adt == jnp.bfloat16 else 16

    @functools.partial(
        pl.kernel,
        out_type=jax.ShapeDtypeStruct((NC, nrows, d), adt),
        mesh=plsc.VectorSubcoreMesh(core_axis_name="c", subcore_axis_name="s"),
        scratch_types=[
            pltpu.VMEM((nch * chunk,), jnp.int32),
            pltpu.VMEM((nch, chunk), jnp.int32),
            pltpu.VMEM((chunk, d), adt),
            pltpu.VMEM((chunk, d), adt),
            pltpu.VMEM_SHARED((nrows, d), adt),
            pltpu.SemaphoreType.DMA,
            pltpu.SemaphoreType.DMA,
        ],
    )
    def scat_kernel(g_hbm, src_hbm, dst_hbm, acc_hbm,
                    src_v, dst_v, buf_a, buf_b, acc_sh, sem_a, sem_b):
        c = lax.axis_index("c")
        s = lax.axis_index("s")
        w = c * NS + s
        pltpu.sync_copy(src_hbm.at[w], src_v)
        pltpu.sync_copy(dst_hbm.at[w], dst_v)

        base = s * rpt

        @pl.when(c == 0)
        def _seed():
            pltpu.sync_copy(g_hbm.at[pl.ds(base, rpt)],
                            acc_sh.at[pl.ds(base, rpt)])

        @pl.when(c != 0)
        def _zfill():
            @pl.loop(0, chunk)
            def _zr(r):
                @pl.loop(0, d // lanes)
                def _zc(i):
                    buf_a[r, pl.ds(i * lanes, lanes)] = jnp.zeros((lanes,), adt)

            @pl.loop(0, nfull)
            def _za(k):
                pltpu.sync_copy(buf_a, acc_sh.at[pl.ds(base + k * chunk, chunk)])

            if rem:
                pltpu.sync_copy(buf_a.at[pl.ds(0, rem)],
                                acc_sh.at[pl.ds(base + nfull * chunk, rem)])

        plsc.subcore_barrier()

        def _gather(j, buf, sem):
            return pltpu.make_async_copy(
                g_hbm.at[src_v.at[pl.ds(j * chunk, chunk)]], buf, sem)

        def _scatter(j, buf):
            pltpu.sync_copy(buf, acc_sh.at[dst_v.at[j]], add=True)

        _gather(0, buf_a, sem_a).start()

        @pl.loop(0, nch // 2)
        def _pair(k):
            j = 2 * k

            @pl.when(j + 1 < nch)
            def _():
                _gather(j + 1, buf_b, sem_b).start()

            _gather(j, buf_a, sem_a).wait()
            _scatter(j, buf_a)

            @pl.when(j + 2 < nch)
            def _():
                _gather(j + 2, buf_a, sem_a).start()

            @pl.when(j + 1 < nch)
            def _():
                _gather(j + 1, buf_b, sem_b).wait()
                _scatter(j + 1, buf_b)

        if nch % 2:
            _gather(nch - 1, buf_a, sem_a).wait()
            _scatter(nch - 1, buf_a)

        plsc.subcore_barrier()

        pltpu.sync_copy(acc_sh.at[pl.ds(base, rpt)],
                        acc_hbm.at[c, pl.ds(base, rpt)])

    return scat_kernel(g, src2, dst3)


def _mm_body(x_ref, w_ref, degp_ref, g_ref):
    deg = jnp.sum(degp_ref[...], axis=0) + 1.0
    dinv = lax.rsqrt(deg)
    h = jnp.dot(x_ref[...], w_ref[...], preferred_element_type=jnp.float32)
    g_ref[...] = h * dinv[:, None]


def _tc_matmul(x, W, degp, nr):
    n, d_in = x.shape
    d = W.shape[1]
    br = 512
    return pl.pallas_call(
        _mm_body,
        grid=(pl.cdiv(nr, br),),
        in_specs=[
            pl.BlockSpec((br, d_in), lambda i: (i, 0)),
            pl.BlockSpec((d_in, d), lambda i: (0, 0)),
            pl.BlockSpec((NC, br), lambda i: (0, i)),
        ],
        out_specs=pl.BlockSpec((br, d), lambda i: (i, 0)),
        out_shape=jax.ShapeDtypeStruct((nr, d), jnp.float32),
    )(x, W, degp)


def _tcb_body(accp_ref, degp_ref, b_ref, o_ref):
    deg = jnp.sum(degp_ref[...], axis=0) + 1.0
    dinv = lax.rsqrt(deg)
    acc = accp_ref[0].astype(jnp.float32) + accp_ref[1].astype(jnp.float32)
    z = acc * dinv[:, None] + b_ref[...]
    m = jnp.max(z, axis=1, keepdims=True)
    ez = jnp.exp(z - m)
    o_ref[...] = z - m - jnp.log(jnp.sum(ez, axis=1, keepdims=True))


def _tc_finalize(accp, degp, b, n):
    d = accp.shape[2]
    br = 512
    return pl.pallas_call(
        _tcb_body,
        grid=(pl.cdiv(n, br),),
        in_specs=[
            pl.BlockSpec((NC, br, d), lambda i: (0, i, 0)),
            pl.BlockSpec((NC, br), lambda i: (0, i)),
            pl.BlockSpec((1, d), lambda i: (0, 0)),
        ],
        out_specs=pl.BlockSpec((br, d), lambda i: (i, 0)),
        out_shape=jax.ShapeDtypeStruct((n, d), jnp.float32),
    )(accp, degp, b.reshape(1, d))


def kernel(x, edge_index, W, b):
    n, d_in = x.shape
    d = W.shape[1]
    e = edge_index.shape[1]
    src = edge_index[0]
    dst = edge_index[1]

    chunk_d, nch_d, pad_d = _plan_edges(e)
    nrows_d = _round_up(n, 256)
    dst_d = dst
    if pad_d:
        ar = jnp.arange(pad_d, dtype=jnp.int32)
        dst_d = jnp.concatenate([dst, n + (ar % 64)])
        nrows_d = _round_up(n + 64, 256)

    chunk_s = 96
    nch_s = -(-e // (NW * chunk_s))
    pad_s = NW * chunk_s * nch_s - e
    n_junk = 16
    nrows_s = _round_up(n + n_junk, 128)
    src_s, dst_s = src, dst
    if pad_s:
        ar = jnp.arange(pad_s, dtype=jnp.int32)
        src_s = jnp.concatenate([src, ar % n])
        dst_s = jnp.concatenate([dst, n + (ar % n_junk)])

    degp = _sc_degree(dst_d.reshape(NW, nch_d, chunk_d), nrows_d)
    g = _tc_matmul(x, W, degp, nrows_s)
    accp = _sc_scatter(g, src_s.reshape(NW, nch_s * chunk_s),
                       dst_s.reshape(NW, nch_s, chunk_s), nrows_s)
    out = _tc_finalize(accp, degp, b, n)
    return out


# speedup vs baseline: 1.5714x; 1.0170x over previous
import functools

import jax
import jax.numpy as jnp
from jax import lax
from jax.experimental import pallas as pl
from jax.experimental.pallas import tpu as pltpu
from jax.experimental.pallas import tpu_sc as plsc

NC = 2
NS = 16
NW = NC * NS


def _round_up(a, m):
    return -(-a // m) * m


def _plan_edges(e, max_chunk=128):
    for chunk in range(max_chunk, 0, -8):
        if e % (NW * chunk) == 0:
            return chunk, e // (NW * chunk), 0
    chunk = max_chunk
    nch = -(-e // (NW * chunk))
    return chunk, nch, NW * chunk * nch - e


def _sc_degree(dst3, nrows):
    nw, nch, chunk = dst3.shape
    rpt = nrows // NS

    @functools.partial(
        pl.kernel,
        out_type=jax.ShapeDtypeStruct((NC, nrows), jnp.float32),
        mesh=plsc.VectorSubcoreMesh(core_axis_name="c", subcore_axis_name="s"),
        scratch_types=[
            pltpu.VMEM((nch, chunk), jnp.int32),
            pltpu.VMEM((chunk,), jnp.float32),
            pltpu.VMEM((rpt,), jnp.float32),
            pltpu.VMEM_SHARED((nrows,), jnp.float32),
            pltpu.SemaphoreType.DMA,
        ],
    )
    def deg_kernel(dst_hbm, degp_hbm, idx_v, ones_v, zb_v, deg_sh, sem):
        c = lax.axis_index("c")
        s = lax.axis_index("s")
        w = c * NS + s
        idx_cp = pltpu.make_async_copy(dst_hbm.at[w], idx_v, sem)
        idx_cp.start()

        @pl.loop(0, chunk // 16)
        def _ones(i):
            ones_v[pl.ds(i * 16, 16)] = jnp.ones((16,), jnp.float32)

        @pl.loop(0, rpt // 16)
        def _zb(i):
            zb_v[pl.ds(i * 16, 16)] = jnp.zeros((16,), jnp.float32)

        pltpu.sync_copy(zb_v, deg_sh.at[pl.ds(s * rpt, rpt)])
        idx_cp.wait()
        plsc.subcore_barrier()

        grp = 5 if nch % 5 == 0 else 1

        @pl.loop(0, nch // grp)
        def _scat(gi):
            for u in range(grp):
                dsc = pltpu.make_async_copy(
                    ones_v, deg_sh.at[idx_v.at[gi * grp + u]], sem)
                dsc.start(add=True)
            for u in range(grp):
                pltpu.make_async_copy(
                    ones_v, deg_sh.at[idx_v.at[gi * grp + u]], sem).wait()

        plsc.subcore_barrier()
        pltpu.sync_copy(deg_sh.at[pl.ds(s * rpt, rpt)],
                        degp_hbm.at[c, pl.ds(s * rpt, rpt)])

    return deg_kernel(dst3)


def _sc_scatter(g, src2, dst3, nrows):
    nw, nch, chunk = dst3.shape
    d = g.shape[1]
    adt = g.dtype
    rpt = nrows // NS
    nfull = rpt // chunk
    rem = rpt % chunk
    lanes = 32 if adt == jnp.bfloat16 else 16

    @functools.partial(
        pl.kernel,
        out_type=jax.ShapeDtypeStruct((NC, nrows, d), adt),
        mesh=plsc.VectorSubcoreMesh(core_axis_name="c", subcore_axis_name="s"),
        scratch_types=[
            pltpu.VMEM((nch * chunk,), jnp.int32),
            pltpu.VMEM((nch, chunk), jnp.int32),
            pltpu.VMEM((chunk, d), adt),
            pltpu.VMEM((chunk, d), adt),
            pltpu.VMEM_SHARED((nrows, d), adt),
            pltpu.SemaphoreType.DMA,
            pltpu.SemaphoreType.DMA,
        ],
    )
    def scat_kernel(g_hbm, src_hbm, dst_hbm, acc_hbm,
                    src_v, dst_v, buf_a, buf_b, acc_sh, sem_a, sem_b):
        c = lax.axis_index("c")
        s = lax.axis_index("s")
        w = c * NS + s
        idx_cp_s = pltpu.make_async_copy(src_hbm.at[w], src_v, sem_a)
        idx_cp_d = pltpu.make_async_copy(dst_hbm.at[w], dst_v, sem_b)
        idx_cp_s.start()
        idx_cp_d.start()

        base = s * rpt

        @pl.when(c == 0)
        def _seed():
            pltpu.sync_copy(g_hbm.at[pl.ds(base, rpt)],
                            acc_sh.at[pl.ds(base, rpt)])

        @pl.when(c != 0)
        def _zfill():
            @pl.loop(0, chunk)
            def _zr(r):
                @pl.loop(0, d // lanes)
                def _zc(i):
                    buf_a[r, pl.ds(i * lanes, lanes)] = jnp.zeros((lanes,), adt)

            @pl.loop(0, nfull)
            def _za(k):
                pltpu.sync_copy(buf_a, acc_sh.at[pl.ds(base + k * chunk, chunk)])

            if rem:
                pltpu.sync_copy(buf_a.at[pl.ds(0, rem)],
                                acc_sh.at[pl.ds(base + nfull * chunk, rem)])

        def _gather(j, buf, sem):
            return pltpu.make_async_copy(
                g_hbm.at[src_v.at[pl.ds(j * chunk, chunk)]], buf, sem)

        def _scatter(j, buf):
            pltpu.sync_copy(buf, acc_sh.at[dst_v.at[j]], add=True)

        idx_cp_s.wait()
        idx_cp_d.wait()
        _gather(0, buf_a, sem_a).start()
        plsc.subcore_barrier()

        @pl.loop(0, nch // 2)
        def _pair(k):
            j = 2 * k

            @pl.when(j + 1 < nch)
            def _():
                _gather(j + 1, buf_b, sem_b).start()

            _gather(j, buf_a, sem_a).wait()
            _scatter(j, buf_a)

            @pl.when(j + 2 < nch)
            def _():
                _gather(j + 2, buf_a, sem_a).start()

            @pl.when(j + 1 < nch)
            def _():
                _gather(j + 1, buf_b, sem_b).wait()
                _scatter(j + 1, buf_b)

        if nch % 2:
            _gather(nch - 1, buf_a, sem_a).wait()
            _scatter(nch - 1, buf_a)

        plsc.subcore_barrier()

        pltpu.sync_copy(acc_sh.at[pl.ds(base, rpt)],
                        acc_hbm.at[c, pl.ds(base, rpt)])

    return scat_kernel(g, src2, dst3)


def _mm_body(x_ref, w_ref, degp_ref, g_ref):
    deg = jnp.sum(degp_ref[...], axis=0) + 1.0
    dinv = lax.rsqrt(deg)
    h = jnp.dot(x_ref[...], w_ref[...], preferred_element_type=jnp.float32)
    g_ref[...] = h * dinv[:, None]


def _tc_matmul(x, W, degp, nr):
    n, d_in = x.shape
    d = W.shape[1]
    br = 512
    return pl.pallas_call(
        _mm_body,
        grid=(pl.cdiv(nr, br),),
        in_specs=[
            pl.BlockSpec((br, d_in), lambda i: (i, 0)),
            pl.BlockSpec((d_in, d), lambda i: (0, 0)),
            pl.BlockSpec((NC, br), lambda i: (0, i)),
        ],
        out_specs=pl.BlockSpec((br, d), lambda i: (i, 0)),
        out_shape=jax.ShapeDtypeStruct((nr, d), jnp.float32),
    )(x, W, degp)


def _tcb_body(accp_ref, degp_ref, b_ref, o_ref):
    deg = jnp.sum(degp_ref[...], axis=0) + 1.0
    dinv = lax.rsqrt(deg)
    acc = accp_ref[0].astype(jnp.float32) + accp_ref[1].astype(jnp.float32)
    z = acc * dinv[:, None] + b_ref[...]
    m = jnp.max(z, axis=1, keepdims=True)
    ez = jnp.exp(z - m)
    o_ref[...] = z - m - jnp.log(jnp.sum(ez, axis=1, keepdims=True))


def _tc_finalize(accp, degp, b, n):
    d = accp.shape[2]
    br = 512
    return pl.pallas_call(
        _tcb_body,
        grid=(pl.cdiv(n, br),),
        in_specs=[
            pl.BlockSpec((NC, br, d), lambda i: (0, i, 0)),
            pl.BlockSpec((NC, br), lambda i: (0, i)),
            pl.BlockSpec((1, d), lambda i: (0, 0)),
        ],
        out_specs=pl.BlockSpec((br, d), lambda i: (i, 0)),
        out_shape=jax.ShapeDtypeStruct((n, d), jnp.float32),
    )(accp, degp, b.reshape(1, d))


def kernel(x, edge_index, W, b):
    n, d_in = x.shape
    d = W.shape[1]
    e = edge_index.shape[1]
    src = edge_index[0]
    dst = edge_index[1]

    chunk_d, nch_d, pad_d = _plan_edges(e)
    nrows_d = _round_up(n, 256)
    dst_d = dst
    if pad_d:
        ar = jnp.arange(pad_d, dtype=jnp.int32)
        dst_d = jnp.concatenate([dst, n + (ar % 64)])
        nrows_d = _round_up(n + 64, 256)

    chunk_s = 96
    nch_s = -(-e // (NW * chunk_s))
    pad_s = NW * chunk_s * nch_s - e
    n_junk = 16
    nrows_s = _round_up(n + n_junk, 128)
    src_s, dst_s = src, dst
    if pad_s:
        ar = jnp.arange(pad_s, dtype=jnp.int32)
        src_s = jnp.concatenate([src, ar % n])
        dst_s = jnp.concatenate([dst, n + (ar % n_junk)])

    degp = _sc_degree(dst_d.reshape(NW, nch_d, chunk_d), nrows_d)
    g = _tc_matmul(x, W, degp, nrows_s)
    accp = _sc_scatter(g, src_s.reshape(NW, nch_s * chunk_s),
                       dst_s.reshape(NW, nch_s, chunk_s), nrows_s)
    out = _tc_finalize(accp, degp, b, n)
    return out
